# baseline (speedup 1.0000x reference)
import functools

import jax
import jax.numpy as jnp
from jax.experimental import pallas as pl
from jax.experimental.pallas import tpu as pltpu


_EPS = 1e-5


def _split_hi_lo(w):
    hi = w.astype(jnp.bfloat16)
    lo = (w - hi.astype(jnp.float32)).astype(jnp.bfloat16)
    return hi, lo


def _body(T, B, HW, C, HID, E,
          x_ref, rwh_ref, rwl_ref, rg_ref, rbb_ref,
          w1h_ref, w1l_ref, wc2_ref,
          g1_ref, bb1_ref, g2_ref, bb2_ref,
          tau_ref, o_ref, wgt_ref):
    e = pl.program_id(0)
    N = T * B
    ROWS = N * HW
    HH = HID // 2
    xs = x_ref[...]

    @pl.when(e == 0)
    def _router():
        xr = xs.reshape(T, B * HW, C)
        v = jnp.zeros_like(xr[0])
        spikes = []
        for t in range(T):
            v = v + (xr[t] - v) / 2.0
            s = (v - 1.0 >= 0.0).astype(jnp.float32)
            spikes.append(s)
            v = v * (1.0 - s)
        S = jnp.concatenate(spikes, axis=0).astype(jnp.bfloat16)
        y = (jax.lax.dot_general(S, rwh_ref[...], (((1,), (0,)), ((), ())),
                                 preferred_element_type=jnp.float32)
             + jax.lax.dot_general(S, rwl_ref[...], (((1,), (0,)), ((), ())),
                                   preferred_element_type=jnp.float32))
        m = jnp.mean(y, axis=0, keepdims=True)
        var = jnp.mean(y * y, axis=0, keepdims=True) - m * m
        ybn = (y - m) * (rg_ref[...] * jax.lax.rsqrt(var + _EPS)) + rbb_ref[...]
        logits = jnp.mean(ybn.reshape(N, HW, E), axis=1)
        mx = jnp.max(logits, axis=1, keepdims=True)
        ex = jnp.exp(logits - mx)
        p = ex / jnp.sum(ex, axis=1, keepdims=True)
        iota = jax.lax.broadcasted_iota(jnp.int32, (N, E), 1)
        m1 = jnp.max(p, axis=1, keepdims=True)
        i1 = jnp.min(jnp.where(p == m1, iota, E), axis=1, keepdims=True)
        p2 = jnp.where(iota == i1, -jnp.inf, p)
        m2 = jnp.max(p2, axis=1, keepdims=True)
        i2 = jnp.min(jnp.where(p2 == m2, iota, E), axis=1, keepdims=True)
        tot = m1 + m2
        wgt_ref[...] = (jnp.where(iota == i1, m1 / tot, 0.0)
                        + jnp.where(iota == i2, m2 / tot, 0.0))

    tau = tau_ref[0, 0, 0]
    s1 = (xs >= tau).astype(jnp.bfloat16)

    def _half(h):
        sl = slice(h * HH, (h + 1) * HH)
        y1 = (jax.lax.dot_general(s1, w1h_ref[0][:, sl],
                                  (((1,), (0,)), ((), ())),
                                  preferred_element_type=jnp.float32)
              + jax.lax.dot_general(s1, w1l_ref[0][:, sl],
                                    (((1,), (0,)), ((), ())),
                                    preferred_element_type=jnp.float32))
        m1 = jnp.mean(y1, axis=0, keepdims=True)
        v1 = jnp.mean(y1 * y1, axis=0, keepdims=True) - m1 * m1
        sc1 = g1_ref[0][:, sl] * jax.lax.rsqrt(v1 + _EPS)
        thr1 = m1 + (tau - bb1_ref[0][:, sl]) / sc1
        return (y1 >= thr1).astype(jnp.bfloat16)

    s2a = _half(0)
    s2b = _half(1)
    y2b = (jax.lax.dot_general(s2a, wc2_ref[0][:HH], (((1,), (0,)), ((), ())),
                               preferred_element_type=jnp.float32)
           + jax.lax.dot_general(s2b, wc2_ref[0][HH:], (((1,), (0,)), ((), ())),
                                 preferred_element_type=jnp.float32))
    y2 = y2b[:, :C] + y2b[:, C:]
    m2 = jnp.mean(y2, axis=0, keepdims=True)
    v2 = jnp.mean(y2 * y2, axis=0, keepdims=True) - m2 * m2
    sc2 = g2_ref[0] * jax.lax.rsqrt(v2 + _EPS)
    y2n = (y2 - m2) * sc2 + bb2_ref[0]
    iota = jax.lax.broadcasted_iota(jnp.int32, (N, E), 1)
    we = jnp.sum(wgt_ref[...] * (iota == e).astype(jnp.float32), axis=1)
    contrib = (y2n.reshape(N, HW, C) * we[:, None, None]).reshape(ROWS, C)

    @pl.when(e == 0)
    def _():
        o_ref[...] = xs + contrib

    @pl.when(e != 0)
    def _():
        o_ref[...] = o_ref[...] + contrib


def kernel(x, router_w, router_b, router_bn_g, router_bn_b,
           w1, b1, bn1_g, bn1_b, w2, b2, bn2_g, bn2_b):
    T, B, C, H, W = x.shape
    HW = H * W
    N = T * B
    ROWS = N * HW
    E, HID, _ = w1.shape
    taus = [1.9 + i * (2.1 - 1.9) / (E - 1) for i in range(E)]

    xm = jnp.transpose(x.reshape(T, B, C, HW), (0, 1, 3, 2)).reshape(ROWS, C)

    rwh, rwl = _split_hi_lo(router_w.T)
    w1h, w1l = _split_hi_lo(jnp.transpose(w1, (0, 2, 1)))
    w2h, w2l = _split_hi_lo(jnp.transpose(w2, (0, 2, 1)))
    w2c = jnp.concatenate([w2h, w2l], axis=2)
    del router_b, b1, b2

    tau_arr = jnp.asarray(taus, dtype=jnp.float32).reshape(E, 1, 1)
    fused = pl.pallas_call(
        functools.partial(_body, T, B, HW, C, HID, E),
        grid=(E,),
        in_specs=[
            pl.BlockSpec((ROWS, C), lambda e: (0, 0)),
            pl.BlockSpec((C, E), lambda e: (0, 0)),
            pl.BlockSpec((C, E), lambda e: (0, 0)),
            pl.BlockSpec((1, E), lambda e: (0, 0)),
            pl.BlockSpec((1, E), lambda e: (0, 0)),
            pl.BlockSpec((1, C, HID), lambda e: (e, 0, 0)),
            pl.BlockSpec((1, C, HID), lambda e: (e, 0, 0)),
            pl.BlockSpec((1, HID, 2 * C), lambda e: (e, 0, 0)),
            pl.BlockSpec((1, 1, HID), lambda e: (e, 0, 0)),
            pl.BlockSpec((1, 1, HID), lambda e: (e, 0, 0)),
            pl.BlockSpec((1, 1, C), lambda e: (e, 0, 0)),
            pl.BlockSpec((1, 1, C), lambda e: (e, 0, 0)),
            pl.BlockSpec((1, 1, 1), lambda e: (e, 0, 0)),
        ],
        out_specs=pl.BlockSpec((ROWS, C), lambda e: (0, 0)),
        out_shape=jax.ShapeDtypeStruct((ROWS, C), jnp.float32),
        scratch_shapes=[pltpu.VMEM((N, E), jnp.float32)],
    )
    out = fused(xm, rwh, rwl,
                router_bn_g.reshape(1, E), router_bn_b.reshape(1, E),
                w1h, w1l, w2c,
                bn1_g.reshape(E, 1, HID), bn1_b.reshape(E, 1, HID),
                bn2_g.reshape(E, 1, C), bn2_b.reshape(E, 1, C),
                tau_arr)

    return jnp.transpose(out.reshape(T, B, HW, C), (0, 1, 3, 2)).reshape(
        T, B, C, H, W)


# speedup vs baseline: 1.1528x; 1.1528x over previous
import functools

import jax
import jax.numpy as jnp
from jax.experimental import pallas as pl
from jax.experimental.pallas import tpu as pltpu


_EPS = 1e-5


def _split_hi_lo(w):
    hi = w.astype(jnp.bfloat16)
    lo = (w - hi.astype(jnp.float32)).astype(jnp.bfloat16)
    return hi, lo


def _body(T, B, HW, C, HID, E,
          x_ref, rwh_ref, rwl_ref, rg_ref, rbb_ref,
          w1h_ref, w1l_ref, wc2_ref,
          g1_ref, bb1_ref, g2_ref, bb2_ref,
          tau_ref, o_ref, wgt_ref):
    e = pl.program_id(0)
    N = T * B
    ROWS = N * HW
    xs = x_ref[...]

    @pl.when(e == 0)
    def _router():
        xr = xs.reshape(T, B * HW, C)
        v = jnp.zeros_like(xr[0])
        spikes = []
        for t in range(T):
            v = v + (xr[t] - v) / 2.0
            s = (v - 1.0 >= 0.0).astype(jnp.float32)
            spikes.append(s)
            v = v * (1.0 - s)
        S = jnp.concatenate(spikes, axis=0).astype(jnp.bfloat16)
        y = (jax.lax.dot_general(S, rwh_ref[...], (((1,), (0,)), ((), ())),
                                 preferred_element_type=jnp.float32)
             + jax.lax.dot_general(S, rwl_ref[...], (((1,), (0,)), ((), ())),
                                   preferred_element_type=jnp.float32))
        m = jnp.mean(y, axis=0, keepdims=True)
        var = jnp.mean(y * y, axis=0, keepdims=True) - m * m
        ybn = (y - m) * (rg_ref[...] * jax.lax.rsqrt(var + _EPS)) + rbb_ref[...]
        logits = jnp.mean(ybn.reshape(N, HW, E), axis=1)
        mx = jnp.max(logits, axis=1, keepdims=True)
        ex = jnp.exp(logits - mx)
        p = ex / jnp.sum(ex, axis=1, keepdims=True)
        iota = jax.lax.broadcasted_iota(jnp.int32, (N, E), 1)
        m1 = jnp.max(p, axis=1, keepdims=True)
        i1 = jnp.min(jnp.where(p == m1, iota, E), axis=1, keepdims=True)
        p2 = jnp.where(iota == i1, -jnp.inf, p)
        m2 = jnp.max(p2, axis=1, keepdims=True)
        i2 = jnp.min(jnp.where(p2 == m2, iota, E), axis=1, keepdims=True)
        tot = m1 + m2
        wgt_ref[...] = (jnp.where(iota == i1, m1 / tot, 0.0)
                        + jnp.where(iota == i2, m2 / tot, 0.0))

    tau = tau_ref[0, 0, 0]
    s1 = (xs >= tau).astype(jnp.bfloat16)
    y1 = (jax.lax.dot_general(s1, w1h_ref[0], (((1,), (0,)), ((), ())),
                              preferred_element_type=jnp.float32)
          + jax.lax.dot_general(s1, w1l_ref[0], (((1,), (0,)), ((), ())),
                                preferred_element_type=jnp.float32))
    m1 = jnp.mean(y1, axis=0, keepdims=True)
    v1 = jnp.mean(y1 * y1, axis=0, keepdims=True) - m1 * m1
    sc1 = g1_ref[0] * jax.lax.rsqrt(v1 + _EPS)
    thr1 = m1 + (tau - bb1_ref[0]) / sc1
    s2 = (y1 >= thr1).astype(jnp.bfloat16)
    y2b = jax.lax.dot_general(s2, wc2_ref[0], (((1,), (0,)), ((), ())),
                              preferred_element_type=jnp.float32)
    y2 = y2b[:, :C] + y2b[:, C:]
    m2 = jnp.mean(y2, axis=0, keepdims=True)
    v2 = jnp.mean(y2 * y2, axis=0, keepdims=True) - m2 * m2
    sc2 = g2_ref[0] * jax.lax.rsqrt(v2 + _EPS)
    y2n = (y2 - m2) * sc2 + bb2_ref[0]
    iota = jax.lax.broadcasted_iota(jnp.int32, (N, E), 1)
    we = jnp.sum(wgt_ref[...] * (iota == e).astype(jnp.float32), axis=1)
    contrib = (y2n.reshape(N, HW, C) * we[:, None, None]).reshape(ROWS, C)

    @pl.when(e == 0)
    def _():
        o_ref[...] = xs + contrib

    @pl.when(e != 0)
    def _():
        o_ref[...] = o_ref[...] + contrib


def kernel(x, router_w, router_b, router_bn_g, router_bn_b,
           w1, b1, bn1_g, bn1_b, w2, b2, bn2_g, bn2_b):
    T, B, C, H, W = x.shape
    HW = H * W
    N = T * B
    ROWS = N * HW
    E, HID, _ = w1.shape
    taus = [1.9 + i * (2.1 - 1.9) / (E - 1) for i in range(E)]

    xm = jnp.transpose(x.reshape(T, B, C, HW), (0, 1, 3, 2)).reshape(ROWS, C)

    rwh, rwl = _split_hi_lo(router_w.T)
    w1h, w1l = _split_hi_lo(jnp.transpose(w1, (0, 2, 1)))
    w2h, w2l = _split_hi_lo(jnp.transpose(w2, (0, 2, 1)))
    w2c = jnp.concatenate([w2h, w2l], axis=2)
    del router_b, b1, b2

    tau_arr = jnp.asarray(taus, dtype=jnp.float32).reshape(E, 1, 1)
    fused = pl.pallas_call(
        functools.partial(_body, T, B, HW, C, HID, E),
        grid=(E,),
        in_specs=[
            pl.BlockSpec((ROWS, C), lambda e: (0, 0)),
            pl.BlockSpec((C, E), lambda e: (0, 0)),
            pl.BlockSpec((C, E), lambda e: (0, 0)),
            pl.BlockSpec((1, E), lambda e: (0, 0)),
            pl.BlockSpec((1, E), lambda e: (0, 0)),
            pl.BlockSpec((1, C, HID), lambda e: (e, 0, 0)),
            pl.BlockSpec((1, C, HID), lambda e: (e, 0, 0)),
            pl.BlockSpec((1, HID, 2 * C), lambda e: (e, 0, 0)),
            pl.BlockSpec((1, 1, HID), lambda e: (e, 0, 0)),
            pl.BlockSpec((1, 1, HID), lambda e: (e, 0, 0)),
            pl.BlockSpec((1, 1, C), lambda e: (e, 0, 0)),
            pl.BlockSpec((1, 1, C), lambda e: (e, 0, 0)),
            pl.BlockSpec((1, 1, 1), lambda e: (e, 0, 0)),
        ],
        out_specs=pl.BlockSpec((ROWS, C), lambda e: (0, 0)),
        out_shape=jax.ShapeDtypeStruct((ROWS, C), jnp.float32),
        scratch_shapes=[pltpu.VMEM((N, E), jnp.float32)],
    )
    out = fused(xm, rwh, rwl,
                router_bn_g.reshape(1, E), router_bn_b.reshape(1, E),
                w1h, w1l, w2c,
                bn1_g.reshape(E, 1, HID), bn1_b.reshape(E, 1, HID),
                bn2_g.reshape(E, 1, C), bn2_b.reshape(E, 1, C),
                tau_arr)

    return jnp.transpose(out.reshape(T, B, HW, C), (0, 1, 3, 2)).reshape(
        T, B, C, H, W)


# speedup vs baseline: 1.1690x; 1.0141x over previous
import functools

import jax
import jax.numpy as jnp
from jax.experimental import pallas as pl
from jax.experimental.pallas import tpu as pltpu


_EPS = 1e-5


def _split_hi_lo(w):
    hi = w.astype(jnp.bfloat16)
    lo = (w - hi.astype(jnp.float32)).astype(jnp.bfloat16)
    return hi, lo


def _body(T, B, HW, C, HID, E,
          x_ref, rwh_ref, rwl_ref, rg_ref, rbb_ref,
          w1h_ref, w1l_ref, wc2_ref,
          g1_ref, bb1_ref, g2_ref, bb2_ref,
          tau_ref, o_ref, wgt_ref):
    e = pl.program_id(0)
    N = T * B
    ROWS = N * HW
    xs = x_ref[...]

    @pl.when(e == 0)
    def _router():
        xr = xs.reshape(T, B * HW, C)
        v = jnp.zeros_like(xr[0])
        spikes = []
        for t in range(T):
            v = v + (xr[t] - v) / 2.0
            s = (v - 1.0 >= 0.0).astype(jnp.float32)
            spikes.append(s)
            v = v * (1.0 - s)
        S = jnp.concatenate(spikes, axis=0).astype(jnp.bfloat16)
        y = (jax.lax.dot_general(S, rwh_ref[...], (((1,), (0,)), ((), ())),
                                 preferred_element_type=jnp.float32)
             + jax.lax.dot_general(S, rwl_ref[...], (((1,), (0,)), ((), ())),
                                   preferred_element_type=jnp.float32))
        m = jnp.mean(y, axis=0, keepdims=True)
        var = jnp.mean(y * y, axis=0, keepdims=True) - m * m
        ybn = (y - m) * (rg_ref[...] * jax.lax.rsqrt(var + _EPS)) + rbb_ref[...]
        logits = jnp.mean(ybn.reshape(N, HW, E), axis=1)
        mx = jnp.max(logits, axis=1, keepdims=True)
        ex = jnp.exp(logits - mx)
        p = ex / jnp.sum(ex, axis=1, keepdims=True)
        iota = jax.lax.broadcasted_iota(jnp.int32, (N, E), 1)
        m1 = jnp.max(p, axis=1, keepdims=True)
        i1 = jnp.min(jnp.where(p == m1, iota, E), axis=1, keepdims=True)
        p2 = jnp.where(iota == i1, -jnp.inf, p)
        m2 = jnp.max(p2, axis=1, keepdims=True)
        i2 = jnp.min(jnp.where(p2 == m2, iota, E), axis=1, keepdims=True)
        tot = m1 + m2
        wgt_ref[...] = (jnp.where(iota == i1, m1 / tot, 0.0)
                        + jnp.where(iota == i2, m2 / tot, 0.0))

    tau = tau_ref[0, 0, 0]
    s1 = (xs >= tau).astype(jnp.bfloat16)
    y1 = (jax.lax.dot_general(s1, w1h_ref[0], (((1,), (0,)), ((), ())),
                              preferred_element_type=jnp.float32)
          + jax.lax.dot_general(s1, w1l_ref[0], (((1,), (0,)), ((), ())),
                                preferred_element_type=jnp.float32))
    m1 = jnp.mean(y1, axis=0, keepdims=True)
    v1 = jnp.mean(y1 * y1, axis=0, keepdims=True) - m1 * m1
    sc1 = g1_ref[0] * jax.lax.rsqrt(v1 + _EPS)
    thr1 = m1 + (tau - bb1_ref[0]) / sc1
    s2 = (y1 >= thr1).astype(jnp.bfloat16)
    y2b = jax.lax.dot_general(s2, wc2_ref[0], (((1,), (0,)), ((), ())),
                              preferred_element_type=jnp.float32)
    y2 = y2b[:, :C] + y2b[:, C:]
    m2 = jnp.mean(y2, axis=0, keepdims=True)
    v2 = jnp.mean(y2 * y2, axis=0, keepdims=True) - m2 * m2
    sc2 = g2_ref[0] * jax.lax.rsqrt(v2 + _EPS)
    k2 = m2 * sc2 - bb2_ref[0]
    iota = jax.lax.broadcasted_iota(jnp.int32, (N, E), 1)
    we = jnp.sum(wgt_ref[...] * (iota == e).astype(jnp.float32), axis=1)
    contrib = ((y2 * sc2 - k2).reshape(N, HW, C)
               * we[:, None, None]).reshape(ROWS, C)

    @pl.when(e == 0)
    def _():
        o_ref[...] = xs + contrib

    @pl.when(e != 0)
    def _():
        o_ref[...] = o_ref[...] + contrib


def kernel(x, router_w, router_b, router_bn_g, router_bn_b,
           w1, b1, bn1_g, bn1_b, w2, b2, bn2_g, bn2_b):
    T, B, C, H, W = x.shape
    HW = H * W
    N = T * B
    ROWS = N * HW
    E, HID, _ = w1.shape
    taus = [1.9 + i * (2.1 - 1.9) / (E - 1) for i in range(E)]

    xm = jnp.transpose(x.reshape(T, B, C, HW), (0, 1, 3, 2)).reshape(ROWS, C)

    rwh, rwl = _split_hi_lo(router_w.T)
    w1h, w1l = _split_hi_lo(jnp.transpose(w1, (0, 2, 1)))
    w2h, w2l = _split_hi_lo(jnp.transpose(w2, (0, 2, 1)))
    w2c = jnp.concatenate([w2h, w2l], axis=2)
    del router_b, b1, b2

    tau_arr = jnp.asarray(taus, dtype=jnp.float32).reshape(E, 1, 1)
    fused = pl.pallas_call(
        functools.partial(_body, T, B, HW, C, HID, E),
        grid=(E,),
        in_specs=[
            pl.BlockSpec((ROWS, C), lambda e: (0, 0)),
            pl.BlockSpec((C, E), lambda e: (0, 0)),
            pl.BlockSpec((C, E), lambda e: (0, 0)),
            pl.BlockSpec((1, E), lambda e: (0, 0)),
            pl.BlockSpec((1, E), lambda e: (0, 0)),
            pl.BlockSpec((1, C, HID), lambda e: (e, 0, 0)),
            pl.BlockSpec((1, C, HID), lambda e: (e, 0, 0)),
            pl.BlockSpec((1, HID, 2 * C), lambda e: (e, 0, 0)),
            pl.BlockSpec((1, 1, HID), lambda e: (e, 0, 0)),
            pl.BlockSpec((1, 1, HID), lambda e: (e, 0, 0)),
            pl.BlockSpec((1, 1, C), lambda e: (e, 0, 0)),
            pl.BlockSpec((1, 1, C), lambda e: (e, 0, 0)),
            pl.BlockSpec((1, 1, 1), lambda e: (e, 0, 0)),
        ],
        out_specs=pl.BlockSpec((ROWS, C), lambda e: (0, 0)),
        out_shape=jax.ShapeDtypeStruct((ROWS, C), jnp.float32),
        scratch_shapes=[pltpu.VMEM((N, E), jnp.float32)],
    )
    out = fused(xm, rwh, rwl,
                router_bn_g.reshape(1, E), router_bn_b.reshape(1, E),
                w1h, w1l, w2c,
                bn1_g.reshape(E, 1, HID), bn1_b.reshape(E, 1, HID),
                bn2_g.reshape(E, 1, C), bn2_b.reshape(E, 1, C),
                tau_arr)

    return jnp.transpose(out.reshape(T, B, HW, C), (0, 1, 3, 2)).reshape(
        T, B, C, H, W)


# speedup vs baseline: 1.6265x; 1.3913x over previous
import functools

import jax
import jax.numpy as jnp
from jax.experimental import pallas as pl
from jax.experimental.pallas import tpu as pltpu


_EPS = 1e-5


def _split_hi_lo(w):
    hi = w.astype(jnp.bfloat16)
    lo = (w - hi.astype(jnp.float32)).astype(jnp.bfloat16)
    return hi, lo


def _body(T, B, HW, C, HID, E,
          x_ref, rwh_ref, rwl_ref, rg_ref, rbb_ref,
          w1h_ref, w1l_ref, wc2_ref,
          g1_ref, bb1_ref, g2_ref, bb2_ref,
          tau_ref, o_ref, wgt_ref):
    e = pl.program_id(0)
    N = T * B
    ROWS = N * HW
    xs = x_ref[...]

    @pl.when(e == 0)
    def _router():
        xr = xs.reshape(T, HW * B, C)
        v = jnp.zeros_like(xr[0])
        spikes = []
        for t in range(T):
            v = v + (xr[t] - v) / 2.0
            s = (v - 1.0 >= 0.0).astype(jnp.float32)
            spikes.append(s)
            v = v * (1.0 - s)
        S = jnp.concatenate(spikes, axis=0).astype(jnp.bfloat16)
        y = (jax.lax.dot_general(S, rwh_ref[...], (((1,), (0,)), ((), ())),
                                 preferred_element_type=jnp.float32)
             + jax.lax.dot_general(S, rwl_ref[...], (((1,), (0,)), ((), ())),
                                   preferred_element_type=jnp.float32))
        m = jnp.mean(y, axis=0, keepdims=True)
        var = jnp.mean(y * y, axis=0, keepdims=True) - m * m
        ybn = (y - m) * (rg_ref[...] * jax.lax.rsqrt(var + _EPS)) + rbb_ref[...]
        logits = jnp.mean(ybn.reshape(T, HW, B, E), axis=1).reshape(N, E)
        mx = jnp.max(logits, axis=1, keepdims=True)
        ex = jnp.exp(logits - mx)
        p = ex / jnp.sum(ex, axis=1, keepdims=True)
        iota = jax.lax.broadcasted_iota(jnp.int32, (N, E), 1)
        m1 = jnp.max(p, axis=1, keepdims=True)
        i1 = jnp.min(jnp.where(p == m1, iota, E), axis=1, keepdims=True)
        p2 = jnp.where(iota == i1, -jnp.inf, p)
        m2 = jnp.max(p2, axis=1, keepdims=True)
        i2 = jnp.min(jnp.where(p2 == m2, iota, E), axis=1, keepdims=True)
        tot = m1 + m2
        wgt_ref[...] = (jnp.where(iota == i1, m1 / tot, 0.0)
                        + jnp.where(iota == i2, m2 / tot, 0.0))

    tau = tau_ref[0, 0, 0]
    s1 = (xs >= tau).astype(jnp.bfloat16)
    y1 = (jax.lax.dot_general(s1, w1h_ref[0], (((1,), (0,)), ((), ())),
                              preferred_element_type=jnp.float32)
          + jax.lax.dot_general(s1, w1l_ref[0], (((1,), (0,)), ((), ())),
                                preferred_element_type=jnp.float32))
    m1 = jnp.mean(y1, axis=0, keepdims=True)
    v1 = jnp.mean(y1 * y1, axis=0, keepdims=True) - m1 * m1
    sc1 = g1_ref[0] * jax.lax.rsqrt(v1 + _EPS)
    thr1 = m1 + (tau - bb1_ref[0]) / sc1
    s2 = (y1 >= thr1).astype(jnp.bfloat16)
    y2b = jax.lax.dot_general(s2, wc2_ref[0], (((1,), (0,)), ((), ())),
                              preferred_element_type=jnp.float32)
    y2 = y2b[:, :C] + y2b[:, C:]
    m2 = jnp.mean(y2, axis=0, keepdims=True)
    v2 = jnp.mean(y2 * y2, axis=0, keepdims=True) - m2 * m2
    sc2 = g2_ref[0] * jax.lax.rsqrt(v2 + _EPS)
    k2 = m2 * sc2 - bb2_ref[0]
    iota = jax.lax.broadcasted_iota(jnp.int32, (N, E), 1)
    we = jnp.sum(wgt_ref[...] * (iota == e).astype(jnp.float32), axis=1)
    contrib = ((y2 * sc2 - k2).reshape(T, HW, B, C)
               * we.reshape(T, 1, B, 1)).reshape(ROWS, C)

    @pl.when(e == 0)
    def _():
        o_ref[...] = xs + contrib

    @pl.when(e != 0)
    def _():
        o_ref[...] = o_ref[...] + contrib


def kernel(x, router_w, router_b, router_bn_g, router_bn_b,
           w1, b1, bn1_g, bn1_b, w2, b2, bn2_g, bn2_b):
    T, B, C, H, W = x.shape
    HW = H * W
    N = T * B
    ROWS = N * HW
    E, HID, _ = w1.shape
    taus = [1.9 + i * (2.1 - 1.9) / (E - 1) for i in range(E)]

    xm = jnp.transpose(x.reshape(T, B, C, HW), (0, 3, 1, 2)).reshape(ROWS, C)

    rwh, rwl = _split_hi_lo(router_w.T)
    w1h, w1l = _split_hi_lo(jnp.transpose(w1, (0, 2, 1)))
    w2h, w2l = _split_hi_lo(jnp.transpose(w2, (0, 2, 1)))
    w2c = jnp.concatenate([w2h, w2l], axis=2)
    del router_b, b1, b2

    tau_arr = jnp.asarray(taus, dtype=jnp.float32).reshape(E, 1, 1)
    fused = pl.pallas_call(
        functools.partial(_body, T, B, HW, C, HID, E),
        grid=(E,),
        in_specs=[
            pl.BlockSpec((ROWS, C), lambda e: (0, 0)),
            pl.BlockSpec((C, E), lambda e: (0, 0)),
            pl.BlockSpec((C, E), lambda e: (0, 0)),
            pl.BlockSpec((1, E), lambda e: (0, 0)),
            pl.BlockSpec((1, E), lambda e: (0, 0)),
            pl.BlockSpec((1, C, HID), lambda e: (e, 0, 0)),
            pl.BlockSpec((1, C, HID), lambda e: (e, 0, 0)),
            pl.BlockSpec((1, HID, 2 * C), lambda e: (e, 0, 0)),
            pl.BlockSpec((1, 1, HID), lambda e: (e, 0, 0)),
            pl.BlockSpec((1, 1, HID), lambda e: (e, 0, 0)),
            pl.BlockSpec((1, 1, C), lambda e: (e, 0, 0)),
            pl.BlockSpec((1, 1, C), lambda e: (e, 0, 0)),
            pl.BlockSpec((1, 1, 1), lambda e: (e, 0, 0)),
        ],
        out_specs=pl.BlockSpec((ROWS, C), lambda e: (0, 0)),
        out_shape=jax.ShapeDtypeStruct((ROWS, C), jnp.float32),
        scratch_shapes=[pltpu.VMEM((N, E), jnp.float32)],
    )
    out = fused(xm, rwh, rwl,
                router_bn_g.reshape(1, E), router_bn_b.reshape(1, E),
                w1h, w1l, w2c,
                bn1_g.reshape(E, 1, HID), bn1_b.reshape(E, 1, HID),
                bn2_g.reshape(E, 1, C), bn2_b.reshape(E, 1, C),
                tau_arr)

    return jnp.transpose(out.reshape(T, HW, B, C), (0, 2, 3, 1)).reshape(
        T, B, C, H, W)
